# baseline (speedup 1.0000x reference)
import functools

import jax
import jax.numpy as jnp
from jax import lax
from jax.experimental import pallas as pl
from jax.experimental.pallas import tpu as pltpu
from jax.experimental.pallas import tpu_sc as plsc

_NC = 2
_NS = 16
_NW = _NC * _NS
_C = 128
_DIM = 32


def kernel(input, weight):
    num_idx = input.shape[0] * input.shape[1]
    per_w = num_idx // _NW
    n_chunks = per_w // _C
    idx = input.reshape(num_idx).astype(jnp.int32)

    mesh = plsc.VectorSubcoreMesh(core_axis_name="c", subcore_axis_name="s")

    @functools.partial(
        pl.kernel,
        mesh=mesh,
        out_type=jax.ShapeDtypeStruct((num_idx, _DIM), jnp.float32),
        scratch_types=[
            pltpu.VMEM((_C,), jnp.int32),
            pltpu.VMEM((_C, _DIM), jnp.float32),
            pltpu.SemaphoreType.DMA,
        ],
        compiler_params=pltpu.CompilerParams(use_tc_tiling_on_sc=False),
    )
    def gather_kernel(w_hbm, i_hbm, o_hbm, idx_v, rows_v, sem):
        wid = lax.axis_index("s") * _NC + lax.axis_index("c")
        base = wid * per_w

        @pl.loop(0, n_chunks)
        def _(t):
            off = base + t * _C
            pltpu.sync_copy(i_hbm.at[pl.ds(off, _C)], idx_v)
            pltpu.async_copy(w_hbm.at[idx_v], rows_v, sem).wait()
            pltpu.sync_copy(rows_v, o_hbm.at[pl.ds(off, _C)])

    out = gather_kernel(weight, idx)
    return out.reshape(input.shape + (_DIM,))


# speedup vs baseline: 1.1392x; 1.1392x over previous
import functools

import jax
import jax.numpy as jnp
from jax import lax
from jax.experimental import pallas as pl
from jax.experimental.pallas import tpu as pltpu
from jax.experimental.pallas import tpu_sc as plsc

_NC = 2
_NS = 16
_NW = _NC * _NS
_C = 128
_K = 5
_GC = _K * _C
_NBUF = 4
_DIM = 32


def kernel(input, weight):
    num_idx = input.shape[0] * input.shape[1]
    per_w = num_idx // _NW
    n_rounds = per_w // _GC
    idx = input.reshape(num_idx).astype(jnp.int32)

    mesh = plsc.VectorSubcoreMesh(core_axis_name="c", subcore_axis_name="s")

    @functools.partial(
        pl.kernel,
        mesh=mesh,
        out_type=jax.ShapeDtypeStruct((num_idx, _DIM), jnp.float32),
        scratch_types=[
            pltpu.VMEM((per_w,), jnp.int32),
            pltpu.VMEM((_NBUF, _GC, _DIM), jnp.float32),
            pltpu.SemaphoreType.DMA,
            pltpu.SemaphoreType.DMA((_NBUF,)),
            pltpu.SemaphoreType.DMA((_NBUF,)),
        ],
        compiler_params=pltpu.CompilerParams(use_tc_tiling_on_sc=False),
    )
    def gather_kernel(w_hbm, i_hbm, o_hbm, idx_v, rows_v, isem, gsem, ssem):
        wid = lax.axis_index("s") * _NC + lax.axis_index("c")
        base = wid * per_w
        pltpu.async_copy(i_hbm.at[pl.ds(base, per_w)], idx_v, isem).wait()

        def fire_gathers(p, b):
            for k in range(_K):
                pltpu.async_copy(
                    w_hbm.at[idx_v.at[pl.ds((p * _K + k) * _C, _C)]],
                    rows_v.at[b].at[pl.ds(k * _C, _C)],
                    gsem.at[b],
                )

        def drain_gathers(b):
            pltpu.make_async_copy(
                o_hbm.at[pl.ds(0, _GC)], rows_v.at[b], gsem.at[b]
            ).wait()

        def fire_store(p, b):
            pltpu.async_copy(
                rows_v.at[b], o_hbm.at[pl.ds(base + p * _GC, _GC)], ssem.at[b]
            )

        def drain_store(p, b):
            pltpu.make_async_copy(
                rows_v.at[b], o_hbm.at[pl.ds(base + p * _GC, _GC)], ssem.at[b]
            ).wait()

        fire_gathers(0, 0)
        fire_gathers(1, 1)

        @pl.loop(0, n_rounds, step=_NBUF)
        def _(t):
            for j in range(_NBUF):
                p = t + j
                drain_gathers(j)
                fire_store(p, j)

                @pl.when(p >= 2)
                def _():
                    drain_store(p - 2, (j + 2) % _NBUF)

                @pl.when(p + 2 < n_rounds)
                def _():
                    fire_gathers(p + 2, (j + 2) % _NBUF)

        drain_store(n_rounds - 2, (n_rounds - 2) % _NBUF)
        drain_store(n_rounds - 1, (n_rounds - 1) % _NBUF)

    out = gather_kernel(weight, idx)
    return out.reshape(input.shape + (_DIM,))


# speedup vs baseline: 1.6443x; 1.4433x over previous
import functools

import jax
import jax.numpy as jnp
from jax import lax
from jax.experimental import pallas as pl
from jax.experimental.pallas import tpu as pltpu
from jax.experimental.pallas import tpu_sc as plsc

_NC = 2
_NS = 16
_NW = _NC * _NS
_C = 128
_DIM = 32
_NBUF = 4


def kernel(input, weight):
    batch, seq = input.shape
    n_bt = batch // _C
    n_j = n_bt // _NW
    n_items = seq * n_j
    n_dt = _DIM // 8

    idx_t = (
        jnp.transpose(input.astype(jnp.int32))
        .reshape(seq, n_j, _NW, _C)
    )

    mesh = plsc.VectorSubcoreMesh(core_axis_name="c", subcore_axis_name="s")

    @functools.partial(
        pl.kernel,
        mesh=mesh,
        out_type=jax.ShapeDtypeStruct((seq, n_dt, n_bt, 8, _C), jnp.float32),
        scratch_types=[
            pltpu.VMEM((seq, n_j, _C), jnp.int32),
            pltpu.VMEM((_NBUF, _C, _DIM), jnp.float32),
            pltpu.VMEM((_NBUF, n_dt, 8, _C), jnp.float32),
            pltpu.SemaphoreType.DMA,
            pltpu.SemaphoreType.DMA((_NBUF,)),
            pltpu.SemaphoreType.DMA((_NBUF,)),
        ],
        compiler_params=pltpu.CompilerParams(
            use_tc_tiling_on_sc=False, needs_layout_passes=False
        ),
    )
    def gather_kernel(w_hbm, i_hbm, o_hbm, idx_v, rows_v, t_v, isem, gsem, ssem):
        wid = lax.axis_index("s") * _NC + lax.axis_index("c")
        pltpu.async_copy(i_hbm.at[:, :, wid, :], idx_v, isem).wait()

        bi_vecs = [lax.iota(jnp.int32, 16) + 16 * q for q in range(8)]
        d_vecs = [jnp.full((16,), d, jnp.int32) for d in range(_DIM)]

        def fire_gather(s, jc, slot):
            pltpu.async_copy(
                w_hbm.at[idx_v.at[s, jc]], rows_v.at[slot], gsem.at[slot]
            )

        def wait_gather(s, jc, slot):
            pltpu.make_async_copy(
                w_hbm.at[idx_v.at[s, jc]], rows_v.at[slot], gsem.at[slot]
            ).wait()

        def fire_store(s, jc, slot):
            bt = jc * _NW + wid
            pltpu.async_copy(
                t_v.at[slot], o_hbm.at[s, :, bt], ssem.at[slot]
            )

        def drain_store(s, jc, slot):
            bt = jc * _NW + wid
            pltpu.make_async_copy(
                t_v.at[slot], o_hbm.at[s, :, bt], ssem.at[slot]
            ).wait()

        def transpose(slot):
            for d in range(_DIM):
                for q in range(8):
                    x = plsc.load_gather(
                        rows_v.at[slot], [bi_vecs[q], d_vecs[d]]
                    )
                    t_v[slot, d // 8, d % 8, pl.ds(16 * q, 16)] = x

        for i in range(_NBUF - 1):
            fire_gather(i >> 2, i & 3, i)

        @pl.loop(0, n_items, step=_NBUF)
        def _(t):
            s = t >> 2
            for j in range(_NBUF):
                i = t + j

                @pl.when(i >= _NBUF)
                def _():
                    drain_store(s - 1, j, j)

                wait_gather(s, j, j)
                transpose(j)
                fire_store(s, j, j)

                i3 = i + _NBUF - 1
                s3 = i3 >> 2
                jc3 = (j + _NBUF - 1) % _NBUF

                @pl.when(i3 < n_items)
                def _():
                    fire_gather(s3, jc3, (j + _NBUF - 1) % _NBUF)

        for j in range(_NBUF):
            drain_store(seq - 1, j, j)

    out5 = gather_kernel(weight, idx_t)
    out = jnp.transpose(out5, (2, 4, 0, 1, 3)).reshape(batch, seq, _DIM)
    return out


# speedup vs baseline: 2.6691x; 1.6233x over previous
import functools

import jax
import jax.numpy as jnp
from jax import lax
from jax.experimental import pallas as pl
from jax.experimental.pallas import tpu as pltpu
from jax.experimental.pallas import tpu_sc as plsc

_NC = 2
_NS = 16
_NW = _NC * _NS
_C = 128
_DIM = 32
_NBUF = 4


def kernel(input, weight):
    batch, seq = input.shape
    n_bt = batch // _C
    n_j = n_bt // _NW
    n_items = seq * n_j
    n_dt = _DIM // 8

    idx_t = (
        jnp.transpose(input.astype(jnp.int32))
        .reshape(seq, n_j, _NW, _C)
    )

    mesh = plsc.VectorSubcoreMesh(core_axis_name="c", subcore_axis_name="s")

    @functools.partial(
        pl.kernel,
        mesh=mesh,
        out_type=jax.ShapeDtypeStruct((seq, n_dt, n_bt, 8, _C), jnp.float32),
        scratch_types=[
            pltpu.VMEM((seq, n_j, _C), jnp.int32),
            pltpu.VMEM((_NBUF, _C, _DIM), jnp.float32),
            pltpu.VMEM((_NBUF, _DIM, _C), jnp.float32),
            pltpu.SemaphoreType.DMA,
            pltpu.SemaphoreType.DMA((_NBUF,)),
            pltpu.SemaphoreType.DMA((_NBUF,)),
        ],
        compiler_params=pltpu.CompilerParams(
            use_tc_tiling_on_sc=False, needs_layout_passes=False
        ),
    )
    def gather_kernel(w_hbm, i_hbm, o_hbm, idx_v, rows_v, t_v, isem, gsem, ssem):
        wid = lax.axis_index("s") * _NC + lax.axis_index("c")
        pltpu.async_copy(i_hbm.at[:, :, wid, :], idx_v, isem).wait()

        iota = lax.iota(jnp.int32, 16)
        perm = [(iota + k) % 16 for k in range(16)]
        d_vecs = [perm[k] + d0 for d0 in range(0, _DIM, 16) for k in range(16)]

        def fire_gather(s, jc, slot):
            pltpu.async_copy(
                w_hbm.at[idx_v.at[s, jc]], rows_v.at[slot], gsem.at[slot]
            )

        def wait_gather(s, jc, slot):
            pltpu.make_async_copy(
                w_hbm.at[idx_v.at[s, jc]], rows_v.at[slot], gsem.at[slot]
            ).wait()

        def fire_store(s, jc, slot):
            bt = jc * _NW + wid
            for dt in range(n_dt):
                pltpu.async_copy(
                    t_v.at[slot, pl.ds(8 * dt, 8)],
                    o_hbm.at[s, dt, bt],
                    ssem.at[slot],
                )

        def drain_store(s, jc, slot):
            bt = jc * _NW + wid
            for dt in range(n_dt):
                pltpu.make_async_copy(
                    t_v.at[slot, pl.ds(8 * dt, 8)],
                    o_hbm.at[s, dt, bt],
                    ssem.at[slot],
                ).wait()

        def transpose(slot):
            @pl.loop(0, _C // 16)
            def _(bq):
                bvec = iota + bq * 16
                for dk in range(_DIM):
                    x = plsc.load_gather(rows_v.at[slot], [bvec, d_vecs[dk]])
                    plsc.store_scatter(t_v.at[slot], [d_vecs[dk], bvec], x)

        for i in range(_NBUF - 1):
            fire_gather(i >> 2, i & 3, i)

        @pl.loop(0, n_items, step=_NBUF)
        def _(t):
            s = t >> 2
            for j in range(_NBUF):
                i = t + j

                @pl.when(i >= _NBUF)
                def _():
                    drain_store(s - 1, j, j)

                wait_gather(s, j, j)
                transpose(j)
                fire_store(s, j, j)

                i3 = i + _NBUF - 1
                s3 = i3 >> 2
                jc3 = (j + _NBUF - 1) % _NBUF

                @pl.when(i3 < n_items)
                def _():
                    fire_gather(s3, jc3, (j + _NBUF - 1) % _NBUF)

        for j in range(_NBUF):
            drain_store(seq - 1, j, j)

    out5 = gather_kernel(weight, idx_t)
    out = jnp.transpose(out5, (2, 4, 0, 1, 3)).reshape(batch, seq, _DIM)
    return out


# speedup vs baseline: 3.8067x; 1.4262x over previous
import functools

import jax
import jax.numpy as jnp
from jax import lax
from jax.experimental import pallas as pl
from jax.experimental.pallas import tpu as pltpu
from jax.experimental.pallas import tpu_sc as plsc

_NC = 2
_NS = 16
_NW = _NC * _NS
_C = 128
_DIM = 32
_NBUF = 4


def _reformat_weight(weight, mesh):
    v_rows, dim = weight.shape
    assert dim == _DIM
    n_lt_full = v_rows // _C
    tail = v_rows - n_lt_full * _C
    n_rounds = -(-n_lt_full // _NW)
    n_rounds = -(-n_rounds // _NBUF) * _NBUF
    w_t = jnp.transpose(weight)
    w_tail = jnp.transpose(
        jnp.pad(weight[n_lt_full * _C:], ((0, _C - tail), (0, 0)))
    ) if tail else jnp.zeros((dim, _C), weight.dtype)

    @functools.partial(
        pl.kernel,
        mesh=mesh,
        out_type=jax.ShapeDtypeStruct((v_rows * _DIM // _C, _C), jnp.float32),
        scratch_types=[
            pltpu.VMEM((_NBUF, _DIM, _C), jnp.float32),
            pltpu.VMEM((_NBUF, _DIM, _C), jnp.float32),
            pltpu.SemaphoreType.DMA((_NBUF,)),
            pltpu.SemaphoreType.DMA((_NBUF,)),
        ],
        compiler_params=pltpu.CompilerParams(needs_layout_passes=False),
    )
    def reformat_kernel(wt_hbm, wtail_hbm, stg_hbm, src_v, dst_v, lsem, ssem):
        wid = lax.axis_index("s") * _NC + lax.axis_index("c")

        iota = lax.iota(jnp.int32, 16)
        qbase = lax.shift_right_logical(iota, 2)
        perm = [(iota + k) % 16 for k in range(16)]
        d_vecs = [perm[k] + d0 for d0 in range(0, _DIM, 16) for k in range(16)]
        l_vecs = [(iota % 4) * _DIM + d_vecs[dk] for dk in range(_DIM)]

        def lt_of(m):
            return m * _NW + wid

        def fire_load(m, slot):
            pltpu.async_copy(
                wt_hbm.at[:, pl.ds(lt_of(m) * _C, _C)],
                src_v.at[slot],
                lsem.at[slot],
            )

        def wait_load(m, slot):
            pltpu.make_async_copy(
                wt_hbm.at[:, pl.ds(lt_of(m) * _C, _C)],
                src_v.at[slot],
                lsem.at[slot],
            ).wait()

        def fire_store(m, slot):
            pltpu.async_copy(
                dst_v.at[slot],
                stg_hbm.at[pl.ds(lt_of(m) * _DIM, _DIM)],
                ssem.at[slot],
            )

        def drain_store(m, slot):
            pltpu.make_async_copy(
                dst_v.at[slot],
                stg_hbm.at[pl.ds(lt_of(m) * _DIM, _DIM)],
                ssem.at[slot],
            ).wait()

        def transpose(slot, n_bq):
            @pl.loop(0, n_bq)
            def _(bq):
                bvec = iota + bq * 16
                qvec = qbase + bq * 4
                for dk in range(_DIM):
                    x = plsc.load_gather(src_v.at[slot], [d_vecs[dk], bvec])
                    plsc.store_scatter(dst_v.at[slot], [qvec, l_vecs[dk]], x)

        if tail:
            @pl.when(wid == 0)
            def _():
                pltpu.async_copy(wtail_hbm, src_v.at[0], lsem.at[0]).wait()
                transpose(0, _C // 16)
                n_q = tail * _DIM // _C
                pltpu.async_copy(
                    dst_v.at[0, pl.ds(0, n_q)],
                    stg_hbm.at[pl.ds(n_lt_full * _DIM, n_q)],
                    ssem.at[0],
                ).wait()

        def guarded(m, slot, op):
            @pl.when(jnp.logical_and(m >= 0, lt_of(m) < n_lt_full))
            def _():
                op(m, slot)

        for m in range(_NBUF - 1):
            guarded(m, m, fire_load)

        @pl.loop(0, n_rounds, step=_NBUF)
        def _(t):
            for j in range(_NBUF):
                m = t + j
                guarded(m - _NBUF, j, drain_store)
                guarded(m, j, wait_load)

                @pl.when(lt_of(m) < n_lt_full)
                def _():
                    transpose(j, _C // 16)

                guarded(m, j, fire_store)
                guarded(m + _NBUF - 1, (j + _NBUF - 1) % _NBUF, fire_load)

        for j in range(_NBUF):
            guarded(n_rounds - _NBUF + j, j, drain_store)

    return reformat_kernel(w_t, w_tail)


def kernel(input, weight):
    batch, seq = input.shape
    n_bt = batch // _C
    n_j = n_bt // _NW
    n_items = seq * n_j
    n_dt = _DIM // 8

    idx_t = (
        jnp.transpose(input.astype(jnp.int32))
        .reshape(seq, n_j, _NW, _C)
    )

    mesh = plsc.VectorSubcoreMesh(core_axis_name="c", subcore_axis_name="s")

    weight = _reformat_weight(weight, mesh).reshape(weight.shape)

    @functools.partial(
        pl.kernel,
        mesh=mesh,
        out_type=jax.ShapeDtypeStruct((seq, n_dt, n_bt, 8, _C), jnp.float32),
        scratch_types=[
            pltpu.VMEM((seq, n_j, _C), jnp.int32),
            pltpu.VMEM((_NBUF, _C, _DIM), jnp.float32),
            pltpu.VMEM((_NBUF, _DIM, _C), jnp.float32),
            pltpu.SemaphoreType.DMA,
            pltpu.SemaphoreType.DMA((_NBUF,)),
            pltpu.SemaphoreType.DMA((_NBUF,)),
        ],
        compiler_params=pltpu.CompilerParams(
            use_tc_tiling_on_sc=False, needs_layout_passes=False
        ),
    )
    def gather_kernel(w_hbm, i_hbm, o_hbm, idx_v, rows_v, t_v, isem, gsem, ssem):
        wid = lax.axis_index("s") * _NC + lax.axis_index("c")
        pltpu.async_copy(i_hbm.at[:, :, wid, :], idx_v, isem).wait()

        iota = lax.iota(jnp.int32, 16)
        perm = [(iota + k) % 16 for k in range(16)]
        d_vecs = [perm[k] + d0 for d0 in range(0, _DIM, 16) for k in range(16)]

        def fire_gather(s, jc, slot):
            pltpu.async_copy(
                w_hbm.at[idx_v.at[s, jc]], rows_v.at[slot], gsem.at[slot]
            )

        def wait_gather(s, jc, slot):
            pltpu.make_async_copy(
                w_hbm.at[idx_v.at[s, jc]], rows_v.at[slot], gsem.at[slot]
            ).wait()

        def fire_store(s, jc, slot):
            bt = jc * _NW + wid
            for dt in range(n_dt):
                pltpu.async_copy(
                    t_v.at[slot, pl.ds(8 * dt, 8)],
                    o_hbm.at[s, dt, bt],
                    ssem.at[slot],
                )

        def drain_store(s, jc, slot):
            bt = jc * _NW + wid
            for dt in range(n_dt):
                pltpu.make_async_copy(
                    t_v.at[slot, pl.ds(8 * dt, 8)],
                    o_hbm.at[s, dt, bt],
                    ssem.at[slot],
                ).wait()

        def transpose(slot):
            @pl.loop(0, _C // 16)
            def _(bq):
                bvec = iota + bq * 16
                for dk in range(_DIM):
                    x = plsc.load_gather(rows_v.at[slot], [bvec, d_vecs[dk]])
                    plsc.store_scatter(t_v.at[slot], [d_vecs[dk], bvec], x)

        for i in range(_NBUF - 1):
            fire_gather(i >> 2, i & 3, i)

        @pl.loop(0, n_items, step=_NBUF)
        def _(t):
            s = t >> 2
            for j in range(_NBUF):
                i = t + j

                @pl.when(i >= _NBUF)
                def _():
                    drain_store(s - 1, j, j)

                wait_gather(s, j, j)
                transpose(j)
                fire_store(s, j, j)

                i3 = i + _NBUF - 1
                s3 = i3 >> 2
                jc3 = (j + _NBUF - 1) % _NBUF

                @pl.when(i3 < n_items)
                def _():
                    fire_gather(s3, jc3, (j + _NBUF - 1) % _NBUF)

        for j in range(_NBUF):
            drain_store(seq - 1, j, j)

    out5 = gather_kernel(weight, idx_t)
    out = jnp.transpose(out5, (2, 4, 0, 1, 3)).reshape(batch, seq, _DIM)
    return out


# speedup vs baseline: 4.4806x; 1.1770x over previous
import functools

import jax
import jax.numpy as jnp
from jax import lax
from jax.experimental import pallas as pl
from jax.experimental.pallas import tpu as pltpu
from jax.experimental.pallas import tpu_sc as plsc

_NC = 2
_NS = 16
_NW = _NC * _NS
_C = 128
_DIM = 32
_NBUF = 4


def _reformat_weight(weight, mesh):
    v_rows, dim = weight.shape
    assert dim == _DIM
    n_lt_full = v_rows // _C
    tail = v_rows - n_lt_full * _C
    n_rounds = -(-n_lt_full // _NW)
    n_rounds = -(-n_rounds // _NBUF) * _NBUF
    w_t = jnp.transpose(weight)
    w_tail = jnp.transpose(
        jnp.pad(weight[n_lt_full * _C:], ((0, _C - tail), (0, 0)))
    ) if tail else jnp.zeros((dim, _C), weight.dtype)

    @functools.partial(
        pl.kernel,
        mesh=mesh,
        out_type=jax.ShapeDtypeStruct((v_rows * _DIM // _C, _C), jnp.float32),
        scratch_types=[
            pltpu.VMEM((_NBUF, _DIM, _C), jnp.float32),
            pltpu.VMEM((_NBUF, _DIM, _C), jnp.float32),
            pltpu.SemaphoreType.DMA((_NBUF,)),
            pltpu.SemaphoreType.DMA((_NBUF,)),
        ],
        compiler_params=pltpu.CompilerParams(needs_layout_passes=False),
    )
    def reformat_kernel(wt_hbm, wtail_hbm, stg_hbm, src_v, dst_v, lsem, ssem):
        wid = lax.axis_index("s") * _NC + lax.axis_index("c")

        iota = lax.iota(jnp.int32, 16)
        qbase = lax.shift_right_logical(iota, 2)
        perm = [(iota + k) % 16 for k in range(16)]
        d_vecs = [perm[k] + d0 for d0 in range(0, _DIM, 16) for k in range(16)]
        l_vecs = [(iota % 4) * _DIM + d_vecs[dk] for dk in range(_DIM)]

        def lt_of(m):
            return m * _NW + wid

        def fire_load(m, slot):
            pltpu.async_copy(
                wt_hbm.at[:, pl.ds(lt_of(m) * _C, _C)],
                src_v.at[slot],
                lsem.at[slot],
            )

        def wait_load(m, slot):
            pltpu.make_async_copy(
                wt_hbm.at[:, pl.ds(lt_of(m) * _C, _C)],
                src_v.at[slot],
                lsem.at[slot],
            ).wait()

        def fire_store(m, slot):
            pltpu.async_copy(
                dst_v.at[slot],
                stg_hbm.at[pl.ds(lt_of(m) * _DIM, _DIM)],
                ssem.at[slot],
            )

        def drain_store(m, slot):
            pltpu.make_async_copy(
                dst_v.at[slot],
                stg_hbm.at[pl.ds(lt_of(m) * _DIM, _DIM)],
                ssem.at[slot],
            ).wait()

        def transpose(slot, n_bq):
            @plsc.parallel_loop(0, n_bq)
            def _(bq):
                bvec = iota + bq * 16
                qvec = qbase + bq * 4
                for dk in range(_DIM):
                    x = plsc.load_gather(src_v.at[slot], [d_vecs[dk], bvec])
                    plsc.store_scatter(dst_v.at[slot], [qvec, l_vecs[dk]], x)

        if tail:
            @pl.when(wid == 0)
            def _():
                pltpu.async_copy(wtail_hbm, src_v.at[0], lsem.at[0]).wait()
                transpose(0, _C // 16)
                n_q = tail * _DIM // _C
                pltpu.async_copy(
                    dst_v.at[0, pl.ds(0, n_q)],
                    stg_hbm.at[pl.ds(n_lt_full * _DIM, n_q)],
                    ssem.at[0],
                ).wait()

        def guarded(m, slot, op):
            @pl.when(jnp.logical_and(m >= 0, lt_of(m) < n_lt_full))
            def _():
                op(m, slot)

        for m in range(_NBUF - 1):
            guarded(m, m, fire_load)

        @pl.loop(0, n_rounds, step=_NBUF)
        def _(t):
            for j in range(_NBUF):
                m = t + j
                guarded(m - _NBUF, j, drain_store)
                guarded(m, j, wait_load)

                @pl.when(lt_of(m) < n_lt_full)
                def _():
                    transpose(j, _C // 16)

                guarded(m, j, fire_store)
                guarded(m + _NBUF - 1, (j + _NBUF - 1) % _NBUF, fire_load)

        for j in range(_NBUF):
            guarded(n_rounds - _NBUF + j, j, drain_store)

    return reformat_kernel(w_t, w_tail)


def kernel(input, weight):
    batch, seq = input.shape
    n_bt = batch // _C
    n_j = n_bt // _NW
    n_items = seq * n_j
    n_dt = _DIM // 8

    idx_t = (
        jnp.transpose(input.astype(jnp.int32))
        .reshape(seq, n_j, _NW, _C)
    )

    mesh = plsc.VectorSubcoreMesh(core_axis_name="c", subcore_axis_name="s")

    weight = _reformat_weight(weight, mesh).reshape(weight.shape)

    @functools.partial(
        pl.kernel,
        mesh=mesh,
        out_type=jax.ShapeDtypeStruct((seq, n_dt, n_bt, 8, _C), jnp.float32),
        scratch_types=[
            pltpu.VMEM((seq, n_j, _C), jnp.int32),
            pltpu.VMEM((_NBUF, _C, _DIM), jnp.float32),
            pltpu.VMEM((_NBUF, _DIM, _C), jnp.float32),
            pltpu.SemaphoreType.DMA,
            pltpu.SemaphoreType.DMA((_NBUF,)),
            pltpu.SemaphoreType.DMA((_NBUF,)),
        ],
        compiler_params=pltpu.CompilerParams(
            use_tc_tiling_on_sc=False, needs_layout_passes=False
        ),
    )
    def gather_kernel(w_hbm, i_hbm, o_hbm, idx_v, rows_v, t_v, isem, gsem, ssem):
        wid = lax.axis_index("s") * _NC + lax.axis_index("c")
        pltpu.async_copy(i_hbm.at[:, :, wid, :], idx_v, isem).wait()

        iota = lax.iota(jnp.int32, 16)
        perm = [(iota + k) % 16 for k in range(16)]
        d_vecs = [perm[k] + d0 for d0 in range(0, _DIM, 16) for k in range(16)]

        def fire_gather(s, jc, slot):
            pltpu.async_copy(
                w_hbm.at[idx_v.at[s, jc]], rows_v.at[slot], gsem.at[slot]
            )

        def wait_gather(s, jc, slot):
            pltpu.make_async_copy(
                w_hbm.at[idx_v.at[s, jc]], rows_v.at[slot], gsem.at[slot]
            ).wait()

        def fire_store(s, jc, slot):
            bt = jc * _NW + wid
            for dt in range(n_dt):
                pltpu.async_copy(
                    t_v.at[slot, pl.ds(8 * dt, 8)],
                    o_hbm.at[s, dt, bt],
                    ssem.at[slot],
                )

        def drain_store(s, jc, slot):
            bt = jc * _NW + wid
            for dt in range(n_dt):
                pltpu.make_async_copy(
                    t_v.at[slot, pl.ds(8 * dt, 8)],
                    o_hbm.at[s, dt, bt],
                    ssem.at[slot],
                ).wait()

        def transpose(slot):
            @plsc.parallel_loop(0, _C // 16)
            def _(bq):
                bvec = iota + bq * 16
                for dk in range(_DIM):
                    x = plsc.load_gather(rows_v.at[slot], [bvec, d_vecs[dk]])
                    plsc.store_scatter(t_v.at[slot], [d_vecs[dk], bvec], x)

        for i in range(_NBUF - 1):
            fire_gather(i >> 2, i & 3, i)

        @pl.loop(0, n_items, step=_NBUF)
        def _(t):
            s = t >> 2
            for j in range(_NBUF):
                i = t + j

                @pl.when(i >= _NBUF)
                def _():
                    drain_store(s - 1, j, j)

                wait_gather(s, j, j)
                transpose(j)
                fire_store(s, j, j)

                i3 = i + _NBUF - 1
                s3 = i3 >> 2
                jc3 = (j + _NBUF - 1) % _NBUF

                @pl.when(i3 < n_items)
                def _():
                    fire_gather(s3, jc3, (j + _NBUF - 1) % _NBUF)

        for j in range(_NBUF):
            drain_store(seq - 1, j, j)

    out5 = gather_kernel(weight, idx_t)
    out = jnp.transpose(out5, (2, 4, 0, 1, 3)).reshape(batch, seq, _DIM)
    return out
